# Initial kernel scaffold; baseline (speedup 1.0000x reference)
import functools

import jax
import jax.numpy as jnp
from jax import lax
from jax.experimental import pallas as pl
from jax.experimental.pallas import tpu as pltpu
from jax.experimental.pallas import tpu_sc as plsc

_NC, _NS = 2, 16
_NW = _NC * _NS

_B = 16384 * 50
_D = 64
_BPW = _B // _NW
_CH = 128
_NCH = _BPW // _CH


@functools.partial(
    pl.kernel,
    out_type=jax.ShapeDtypeStruct((_B, _D), jnp.float32),
    mesh=plsc.VectorSubcoreMesh(core_axis_name="c", subcore_axis_name="s"),
    scratch_types=[
        pltpu.VMEM((_BPW,), jnp.int32),
        pltpu.VMEM((_CH, _D), jnp.float32),
        pltpu.SemaphoreType.DMA,
    ],
)
def _lookup(table_hbm, idx_hbm, out_hbm, idx_v, rows_v, sem):
    wid = lax.axis_index("s") * _NC + lax.axis_index("c")
    base = wid * _BPW
    pltpu.sync_copy(idx_hbm.at[pl.ds(base, _BPW)], idx_v)

    @pl.loop(0, _NCH)
    def _step(c):
        off = pl.multiple_of(c * _CH, _CH)
        pltpu.async_copy(
            table_hbm.at[idx_v.at[pl.ds(off, _CH)]], rows_v, sem
        ).wait()
        pltpu.sync_copy(rows_v, out_hbm.at[pl.ds(base + off, _CH)])


def kernel(table, indices):
    idx = indices.reshape(-1).astype(jnp.int32)
    out = _lookup(table, idx)
    return out.reshape(indices.shape + (table.shape[1],))


# speedup vs baseline: 1.6843x; 1.6843x over previous
import functools

import jax
import jax.numpy as jnp
from jax import lax
from jax.experimental import pallas as pl
from jax.experimental.pallas import tpu as pltpu
from jax.experimental.pallas import tpu_sc as plsc

_NC, _NS = 2, 16
_NW = _NC * _NS

_B = 16384 * 50
_D = 64
_BPW = _B // _NW
_CH = 128
_NCH = _BPW // _CH


@functools.partial(
    pl.kernel,
    out_type=jax.ShapeDtypeStruct((_B, _D), jnp.float32),
    mesh=plsc.VectorSubcoreMesh(core_axis_name="c", subcore_axis_name="s"),
    scratch_types=[
        pltpu.VMEM((_BPW,), jnp.int32),
        pltpu.VMEM((_CH, _D), jnp.float32),
        pltpu.SemaphoreType.DMA,
    ],
    compiler_params=pltpu.CompilerParams(use_tc_tiling_on_sc=False),
)
def _lookup(table_hbm, idx_hbm, out_hbm, idx_v, rows_v, sem):
    wid = lax.axis_index("s") * _NC + lax.axis_index("c")
    base = wid * _BPW
    pltpu.sync_copy(idx_hbm.at[pl.ds(base, _BPW)], idx_v)

    @pl.loop(0, _NCH)
    def _step(c):
        off = pl.multiple_of(c * _CH, _CH)
        pltpu.async_copy(
            table_hbm.at[idx_v.at[pl.ds(off, _CH)]], rows_v, sem
        ).wait()
        pltpu.sync_copy(rows_v, out_hbm.at[pl.ds(base + off, _CH)])


def kernel(table, indices):
    idx = indices.reshape(-1).astype(jnp.int32)
    out = _lookup(table, idx)
    return out.reshape(indices.shape + (table.shape[1],))


# speedup vs baseline: 1.8764x; 1.1140x over previous
import functools

import jax
import jax.numpy as jnp
from jax import lax
from jax.experimental import pallas as pl
from jax.experimental.pallas import tpu as pltpu
from jax.experimental.pallas import tpu_sc as plsc

_NC, _NS = 2, 16
_NW = _NC * _NS

_B = 16384 * 50
_D = 64
_BPW = _B // _NW
_CH = 128
_NCH = _BPW // _CH
_NBUF = 8


@functools.partial(
    pl.kernel,
    out_type=jax.ShapeDtypeStruct((_B, _D), jnp.float32),
    mesh=plsc.VectorSubcoreMesh(core_axis_name="c", subcore_axis_name="s"),
    scratch_types=[
        pltpu.VMEM((_BPW,), jnp.int32),
        pltpu.VMEM((_NBUF, _CH, _D), jnp.float32),
        pltpu.SemaphoreType.DMA((_NBUF,)),
        pltpu.SemaphoreType.DMA((_NBUF,)),
    ],
    compiler_params=pltpu.CompilerParams(use_tc_tiling_on_sc=False),
)
def _lookup(table_hbm, idx_hbm, out_hbm, idx_v, rows_v, gsem, ssem):
    wid = lax.axis_index("s") * _NC + lax.axis_index("c")
    base = wid * _BPW
    pltpu.sync_copy(idx_hbm.at[pl.ds(base, _BPW)], idx_v)

    def gather_start(slot, c):
        off = pl.multiple_of(c * _CH, _CH)
        return pltpu.async_copy(
            table_hbm.at[idx_v.at[pl.ds(off, _CH)]],
            rows_v.at[slot],
            gsem.at[slot],
        )

    for b in range(_NBUF - 1):
        gather_start(b, b)

    @pl.loop(0, _NCH, step=_NBUF)
    def _ring(g):
        for b in range(_NBUF):
            c = g + b
            off = pl.multiple_of(c * _CH, _CH)
            pltpu.make_async_copy(
                table_hbm.at[idx_v.at[pl.ds(off, _CH)]],
                rows_v.at[b],
                gsem.at[b],
            ).wait()
            pltpu.async_copy(
                rows_v.at[b], out_hbm.at[pl.ds(base + off, _CH)], ssem.at[b]
            )
            f = c + _NBUF - 1
            fb = (b + _NBUF - 1) % _NBUF

            @pl.when(f < _NCH)
            def _():
                @pl.when(f >= _NBUF)
                def _():
                    pltpu.make_async_copy(
                        rows_v.at[fb],
                        out_hbm.at[pl.ds(base, _CH)],
                        ssem.at[fb],
                    ).wait()

                gather_start(fb, f)

    for b in range(_NBUF):
        pltpu.make_async_copy(
            rows_v.at[b], out_hbm.at[pl.ds(base, _CH)], ssem.at[b]
        ).wait()


def kernel(table, indices):
    idx = indices.reshape(-1).astype(jnp.int32)
    out = _lookup(table, idx)
    return out.reshape(indices.shape + (table.shape[1],))
